# baseline (speedup 1.0000x reference)
import functools

import jax
import jax.numpy as jnp
from jax import lax
from jax.experimental import pallas as pl
from jax.experimental.pallas import tpu as pltpu
from jax.experimental.pallas import tpu_sc as plsc

_LANES = 16


def _mm_body(a_ref, b_ref, o_ref):
    o_ref[...] = jnp.dot(a_ref[...], b_ref[...],
                         preferred_element_type=jnp.float32)


def _matmul(a, b, bm=512, bn=1024):
    m, k = a.shape
    _, n = b.shape
    grid = (n // bn, m // bm)
    return pl.pallas_call(
        _mm_body,
        grid=grid,
        in_specs=[
            pl.BlockSpec((bm, k), lambda ni, mi: (mi, 0)),
            pl.BlockSpec((k, bn), lambda ni, mi: (0, ni)),
        ],
        out_specs=pl.BlockSpec((bm, bn), lambda ni, mi: (mi, ni)),
        out_shape=jax.ShapeDtypeStruct((m, n), jnp.float32),
    )(a, b)


_NW = 32
_RB = 8


def _gather_body(n_items, ncand, rpw, score_hbm, items_hbm, out_hbm,
                 items_v, buf0, buf1, out_v, sem0, sem1):
    cid = lax.axis_index("c")
    sid = lax.axis_index("s")
    wid = sid * 2 + cid
    row0 = wid * rpw
    nidx = rpw * ncand
    nchunk = (ncand + _LANES - 1) // _LANES

    pltpu.sync_copy(items_hbm.at[pl.ds(row0 * ncand, nidx)], items_v)

    bufs = (buf0, buf1)
    sems = (sem0, sem1)
    nblk = rpw // _RB
    handles = {0: pltpu.async_copy(score_hbm.at[pl.ds(row0, _RB)], buf0, sem0)}
    for blk in range(nblk):
        if blk + 1 < nblk:
            handles[blk + 1] = pltpu.async_copy(
                score_hbm.at[pl.ds(row0 + (blk + 1) * _RB, _RB)],
                bufs[(blk + 1) % 2], sems[(blk + 1) % 2])
        handles.pop(blk).wait()
        buf = bufs[blk % 2]

        def row_body(rr, carry, blk=blk, buf=buf):
            rowc = blk * _RB + rr
            rsplat = jnp.full((_LANES,), 0, jnp.int32) + rr
            csplat = jnp.full((_LANES,), 0, jnp.int32) + rowc
            lanes = lax.iota(jnp.int32, _LANES)
            for c in range(nchunk):
                j = c * _LANES + lanes
                addr = jnp.minimum(rowc * ncand + j, nidx - 1)
                cand = plsc.load_gather(items_v, [addr])
                vals = plsc.load_gather(buf, [rsplat, cand])
                if (c + 1) * _LANES <= ncand:
                    plsc.store_scatter(out_v, [csplat, j], vals)
                else:
                    plsc.store_scatter(out_v, [csplat, j], vals,
                                       mask=j < ncand)
            return carry

        lax.fori_loop(0, _RB, row_body, 0)

    pltpu.sync_copy(out_v, out_hbm.at[pl.ds(row0, rpw)])


def _gather(score, items):
    b, ncand = items.shape
    n_items = score.shape[1]
    rpw = b // _NW
    mesh = plsc.VectorSubcoreMesh(core_axis_name="c", subcore_axis_name="s")
    f = pl.kernel(
        functools.partial(_gather_body, n_items, ncand, rpw),
        out_type=jax.ShapeDtypeStruct((b, ncand), jnp.float32),
        mesh=mesh,
        compiler_params=pltpu.CompilerParams(needs_layout_passes=False),
        scratch_types=[
            pltpu.VMEM((rpw * ncand,), jnp.int32),
            pltpu.VMEM((_RB, n_items), jnp.float32),
            pltpu.VMEM((_RB, n_items), jnp.float32),
            pltpu.VMEM((rpw, ncand), jnp.float32),
            pltpu.SemaphoreType.DMA,
            pltpu.SemaphoreType.DMA,
        ],
    )
    return f(score, items.reshape(-1))


def kernel(mat, sim, test_sample):
    items = test_sample[:, 1:]
    b = items.shape[0]
    a_bf = mat[:b].astype(jnp.bfloat16)
    s_bf = sim.astype(jnp.bfloat16)
    score = _matmul(a_bf, s_bf)
    return _gather(score, items)
